# baseline (speedup 1.0000x reference)
import functools

import jax
import jax.numpy as jnp
from jax import lax
from jax.experimental import pallas as pl
from jax.experimental.pallas import tpu as pltpu
from jax.experimental.pallas import tpu_sc as plsc

D = 64
SCALE = 8.0
SUB = 128
GATHERS_PER_CHUNK = 4
C = SUB * GATHERS_PER_CHUNK


def _make_lookup(n_rows: int):
    info = plsc.get_sparse_core_info()
    nc, ns = info.num_cores, info.num_subcores
    nw = nc * ns
    per_w = n_rows // nw
    subs_per_w = per_w // SUB
    n_chunks = per_w // C

    mesh = plsc.VectorSubcoreMesh(core_axis_name="c", subcore_axis_name="s")

    @functools.partial(
        pl.kernel,
        out_type=jax.ShapeDtypeStruct((n_rows, D), jnp.float32),
        mesh=mesh,
        scratch_types=[
            pltpu.VMEM((subs_per_w, SUB), jnp.int32),
            pltpu.VMEM((C, D), jnp.float32),
            pltpu.SemaphoreType.DMA,
        ],
        compiler_params=pltpu.CompilerParams(use_tc_tiling_on_sc=False),
    )
    def lookup(idx_hbm, table_hbm, out_hbm, idx_v, rows_v, sem):
        wid = lax.axis_index("s") * nc + lax.axis_index("c")
        base = wid * per_w
        pltpu.sync_copy(idx_hbm.at[pl.ds(wid * subs_per_w, subs_per_w)], idx_v)

        def chunk_body(g, carry):
            for j in range(GATHERS_PER_CHUNK):
                pltpu.async_copy(
                    table_hbm.at[idx_v.at[g * GATHERS_PER_CHUNK + j]],
                    rows_v.at[pl.ds(j * SUB, SUB)],
                    sem,
                ).wait()

            def scale_row(i, c2):
                for t in range(D // 16):
                    sl = pl.ds(t * 16, 16)
                    rows_v[i, sl] = rows_v[i, sl] * SCALE
                return c2

            lax.fori_loop(0, C, scale_row, None)
            pltpu.sync_copy(rows_v, out_hbm.at[pl.ds(base + g * C, C)])
            return carry

        lax.fori_loop(0, n_chunks, chunk_body, None)

    return lookup


def kernel(input_, table):
    l, b = input_.shape
    n = l * b
    idx2d = input_.reshape(n // SUB, SUB)
    out = _make_lookup(n)(idx2d, table)
    return out.reshape(l, b, D)


# speedup vs baseline: 1.1891x; 1.1891x over previous
import functools

import jax
import jax.numpy as jnp
from jax import lax
from jax.experimental import pallas as pl
from jax.experimental.pallas import tpu as pltpu
from jax.experimental.pallas import tpu_sc as plsc

D = 64
SCALE = 8.0
SUB = 128
GPC = 2
C = SUB * GPC
NBUF = 4


def _make_lookup(n_rows: int):
    info = plsc.get_sparse_core_info()
    nc, ns = info.num_cores, info.num_subcores
    nw = nc * ns
    per_w = n_rows // nw
    subs_per_w = per_w // SUB
    n_chunks = per_w // C

    mesh = plsc.VectorSubcoreMesh(core_axis_name="c", subcore_axis_name="s")

    @functools.partial(
        pl.kernel,
        out_type=jax.ShapeDtypeStruct((n_rows, D), jnp.float32),
        mesh=mesh,
        scratch_types=[
            pltpu.VMEM((subs_per_w, SUB), jnp.int32),
            pltpu.VMEM((NBUF, C, D), jnp.float32),
            pltpu.SemaphoreType.DMA((NBUF,)),
            pltpu.SemaphoreType.DMA((NBUF,)),
        ],
        compiler_params=pltpu.CompilerParams(use_tc_tiling_on_sc=False),
    )
    def lookup(idx_hbm, table_hbm, out_hbm, idx_v, bufs, gsem, ssem):
        wid = lax.axis_index("s") * nc + lax.axis_index("c")
        base = wid * per_w
        pltpu.sync_copy(idx_hbm.at[pl.ds(wid * subs_per_w, subs_per_w)], idx_v)

        def gather(g, b):
            return [
                pltpu.make_async_copy(
                    table_hbm.at[idx_v.at[g * GPC + j]],
                    bufs.at[b].at[pl.ds(j * SUB, SUB)],
                    gsem.at[b],
                )
                for j in range(GPC)
            ]

        def scatter(g, b):
            return pltpu.make_async_copy(
                bufs.at[b], out_hbm.at[pl.ds(base + g * C, C)], ssem.at[b]
            )

        for b in range(NBUF - 1):
            for cp in gather(b, b):
                cp.start()

        def outer(k, carry):
            g0 = k * NBUF
            for b in range(NBUF):
                g = g0 + b
                for cp in gather(g, b):
                    cp.wait()

                def scale_row(i, c2):
                    for t in range(D // 16):
                        sl = pl.ds(t * 16, 16)
                        bufs[b, i, sl] = bufs[b, i, sl] * SCALE
                    return c2

                lax.fori_loop(0, C, scale_row, None)
                scatter(g, b).start()

                pb = (b - 1) % NBUF

                @pl.when(g > 0)
                def _():
                    scatter(g - 1, pb).wait()

                @pl.when(g + NBUF - 1 < n_chunks)
                def _():
                    for cp in gather(g + NBUF - 1, pb):
                        cp.start()

            return carry

        lax.fori_loop(0, n_chunks // NBUF, outer, None)
        scatter(n_chunks - 1, (n_chunks - 1) % NBUF).wait()

    return lookup


def kernel(input_, table):
    l, b = input_.shape
    n = l * b
    idx2d = input_.reshape(n // SUB, SUB)
    out = _make_lookup(n)(idx2d, table)
    return out.reshape(l, b, D)
